# baseline (speedup 1.0000x reference)
import functools

import jax
import jax.numpy as jnp
from jax.experimental import pallas as pl
from jax.experimental.pallas import tpu as pltpu

N_EMBD = 768
NUM_EXPERTS = 8
TOP_K = 2
BLOCK_T = 8192


def _router_block(x_ref, w_ref, w_out, i_out, aux_ref, psum, csum,
                  *, nblocks, ntok):
    step = pl.program_id(0)

    @pl.when(step == 0)
    def _init():
        psum[...] = jnp.zeros_like(psum)
        csum[...] = jnp.zeros_like(csum)

    lg = jax.lax.dot_general(
        w_ref[...], x_ref[...], (((1,), (1,)), ((), ())),
        preferred_element_type=jnp.float32)

    m1 = jnp.max(lg, axis=0, keepdims=True)
    e = jnp.exp(lg - m1)
    denom = jnp.sum(e, axis=0, keepdims=True)
    r = 1.0 / denom
    sub = jax.lax.broadcasted_iota(jnp.int32, lg.shape, 0)
    i1 = jnp.min(jnp.where(lg == m1, sub, NUM_EXPERTS), axis=0, keepdims=True)
    masked = jnp.where(sub == i1, -jnp.inf, lg)
    m2 = jnp.max(masked, axis=0, keepdims=True)
    i2 = jnp.min(jnp.where(masked == m2, sub, NUM_EXPERTS),
                 axis=0, keepdims=True)

    p1 = r
    p2 = jnp.exp(m2 - m1) * r
    ws = 1.0 / (p1 + p2)
    w_out[...] = jnp.concatenate([p1 * ws, p2 * ws], axis=0)
    i_out[...] = jnp.concatenate([i1, i2], axis=0)

    psum[...] += e * r
    csum[...] += jnp.logical_or(sub == i1, sub == i2).astype(jnp.float32)

    @pl.when(step == nblocks - 1)
    def _finish():
        P = jnp.sum(psum[...], axis=1, keepdims=True)
        F = jnp.sum(csum[...], axis=1, keepdims=True)
        aux_ref[0, 0] = (NUM_EXPERTS / (ntok * ntok)) * jnp.sum(P * F)


def kernel(x, W):
    B, T, C = x.shape
    ntok = B * T
    x_flat = x.reshape(ntok, C)
    nblocks = ntok // BLOCK_T

    body = functools.partial(_router_block, nblocks=nblocks, ntok=ntok)

    w_t, i_t, aux = pl.pallas_call(
        body,
        grid=(nblocks,),
        in_specs=[
            pl.BlockSpec((BLOCK_T, N_EMBD), lambda i: (i, 0)),
            pl.BlockSpec((NUM_EXPERTS, N_EMBD), lambda i: (0, 0)),
        ],
        out_specs=[
            pl.BlockSpec((TOP_K, BLOCK_T), lambda i: (0, i)),
            pl.BlockSpec((TOP_K, BLOCK_T), lambda i: (0, i)),
            pl.BlockSpec(memory_space=pltpu.SMEM),
        ],
        out_shape=[
            jax.ShapeDtypeStruct((TOP_K, ntok), jnp.float32),
            jax.ShapeDtypeStruct((TOP_K, ntok), jnp.int32),
            jax.ShapeDtypeStruct((1, 1), jnp.float32),
        ],
        scratch_shapes=[
            pltpu.VMEM((NUM_EXPERTS, BLOCK_T), jnp.float32),
            pltpu.VMEM((NUM_EXPERTS, BLOCK_T), jnp.float32),
        ],
    )(x_flat, W)
    return w_t.T, i_t.T, aux[0, 0]


# speedup vs baseline: 1.0884x; 1.0884x over previous
import functools

import jax
import jax.numpy as jnp
from jax.experimental import pallas as pl
from jax.experimental.pallas import tpu as pltpu

N_EMBD = 768
NUM_EXPERTS = 8
TOP_K = 2
BLOCK_T = 4096


def _router_block(x_ref, w_ref, w_out, i_out, aux_ref, psum, csum,
                  *, nblocks, ntok):
    step = pl.program_id(0)

    @pl.when(step == 0)
    def _init():
        psum[...] = jnp.zeros_like(psum)
        csum[...] = jnp.zeros_like(csum)

    lg = jax.lax.dot_general(
        w_ref[...], x_ref[...], (((1,), (1,)), ((), ())),
        preferred_element_type=jnp.float32)

    m1 = jnp.max(lg, axis=0, keepdims=True)
    e = jnp.exp(lg - m1)
    denom = jnp.sum(e, axis=0, keepdims=True)
    r = 1.0 / denom
    sub = jax.lax.broadcasted_iota(jnp.int32, lg.shape, 0)
    i1 = jnp.min(jnp.where(lg == m1, sub, NUM_EXPERTS), axis=0, keepdims=True)
    masked = jnp.where(sub == i1, -jnp.inf, lg)
    m2 = jnp.max(masked, axis=0, keepdims=True)
    i2 = jnp.min(jnp.where(masked == m2, sub, NUM_EXPERTS),
                 axis=0, keepdims=True)

    p1 = r
    p2 = jnp.exp(m2 - m1) * r
    ws = 1.0 / (p1 + p2)
    w_out[...] = jnp.concatenate([p1 * ws, p2 * ws], axis=0)
    i_out[...] = jnp.concatenate([i1, i2], axis=0)

    psum[...] += e * r
    csum[...] += jnp.logical_or(sub == i1, sub == i2).astype(jnp.float32)

    @pl.when(step == nblocks - 1)
    def _finish():
        P = jnp.sum(psum[...], axis=1, keepdims=True)
        F = jnp.sum(csum[...], axis=1, keepdims=True)
        aux_ref[0, 0] = (NUM_EXPERTS / (ntok * ntok)) * jnp.sum(P * F)


def kernel(x, W):
    B, T, C = x.shape
    ntok = B * T
    x_flat = x.reshape(ntok, C)
    nblocks = ntok // BLOCK_T

    body = functools.partial(_router_block, nblocks=nblocks, ntok=ntok)

    w_t, i_t, aux = pl.pallas_call(
        body,
        grid=(nblocks,),
        in_specs=[
            pl.BlockSpec((BLOCK_T, N_EMBD), lambda i: (i, 0)),
            pl.BlockSpec((NUM_EXPERTS, N_EMBD), lambda i: (0, 0)),
        ],
        out_specs=[
            pl.BlockSpec((TOP_K, BLOCK_T), lambda i: (0, i)),
            pl.BlockSpec((TOP_K, BLOCK_T), lambda i: (0, i)),
            pl.BlockSpec(memory_space=pltpu.SMEM),
        ],
        out_shape=[
            jax.ShapeDtypeStruct((TOP_K, ntok), jnp.float32),
            jax.ShapeDtypeStruct((TOP_K, ntok), jnp.int32),
            jax.ShapeDtypeStruct((1, 1), jnp.float32),
        ],
        scratch_shapes=[
            pltpu.VMEM((NUM_EXPERTS, BLOCK_T), jnp.float32),
            pltpu.VMEM((NUM_EXPERTS, BLOCK_T), jnp.float32),
        ],
    )(x_flat, W)
    return w_t.T, i_t.T, aux[0, 0]
